# baseline (speedup 1.0000x reference)
import functools

import jax
import jax.numpy as jnp
from jax import lax
from jax.experimental import pallas as pl
from jax.experimental.pallas import tpu as pltpu
from jax.experimental.pallas import tpu_sc as plsc

_NC, _NS = 2, 16
_NW = _NC * _NS
_EPS = 1e-12

_CH = 32
_NBUF = 3


def _sc_gather(ids3, table, n, d, n_ch):
  mesh = plsc.VectorSubcoreMesh(core_axis_name="c", subcore_axis_name="s")
  scratch = [pltpu.VMEM((n_ch, _CH), jnp.int32)]
  scratch += [pltpu.VMEM((_CH, d), jnp.float32) for _ in range(_NBUF)]
  scratch += [pltpu.SemaphoreType.DMA for _ in range(2 * _NBUF)]

  @functools.partial(
      pl.kernel,
      mesh=mesh,
      out_type=jax.ShapeDtypeStruct((n, d), jnp.float32),
      scratch_types=scratch,
  )
  def gather_kernel(ids_hbm, tab_hbm, out_hbm, idx_v, *rest):
    bufs = rest[:_NBUF]
    gsems = rest[_NBUF:2 * _NBUF]
    wsems = rest[2 * _NBUF:]
    wid = lax.axis_index("s") * _NC + lax.axis_index("c")
    base = wid * (n_ch * _CH)
    pltpu.sync_copy(ids_hbm.at[wid], idx_v)

    def start_gather(c):
      return pltpu.async_copy(
          tab_hbm.at[idx_v.at[c]], bufs[c % _NBUF], gsems[c % _NBUF])

    def start_write(c):
      return pltpu.async_copy(
          bufs[c % _NBUF], out_hbm.at[pl.ds(base + c * _CH, _CH)],
          wsems[c % _NBUF])

    gathers = [None] * n_ch
    writes = [None] * n_ch
    for c in range(min(_NBUF, n_ch)):
      gathers[c] = start_gather(c)
    for c in range(n_ch):
      if c >= 1 and c - 1 + _NBUF < n_ch:
        writes[c - 1].wait()
        gathers[c - 1 + _NBUF] = start_gather(c - 1 + _NBUF)
      gathers[c].wait()
      writes[c] = start_write(c)
    for c in range(max(0, n_ch - _NBUF), n_ch):
      writes[c].wait()

  return gather_kernel(ids3, table)


def _ln_math(g_ref, tt_ref, pos_ref, tok_ref, gam_ref, bet_ref):
  x = g_ref[...]
  ttf = tt_ref[...]
  tok0 = tok_ref[0, :][None, :]
  tokd = (tok_ref[1, :] - tok_ref[0, :])[None, :]
  x = x + pos_ref[...] + tok0 + ttf * tokd
  mean = jnp.mean(x, axis=1, keepdims=True)
  xc = x - mean
  var = jnp.mean(xc * xc, axis=1, keepdims=True)
  inv = lax.rsqrt(var + _EPS)
  return xc * inv * gam_ref[...] + bet_ref[...]


def _ln_body_first(g_ref, tt_ref, pos_ref, tok_ref, gam_ref, bet_ref, o_ref):
  o_ref[0] = _ln_math(g_ref, tt_ref, pos_ref, tok_ref, gam_ref, bet_ref)


def _ln_body_acc(acc_ref, g_ref, tt_ref, pos_ref, tok_ref, gam_ref, bet_ref,
                 o_ref):
  del acc_ref
  o_ref[0] = _ln_math(g_ref, tt_ref, pos_ref, tok_ref, gam_ref, bet_ref)


def _tc_ln_chunk(acc, bb, gathered, ttf, pos_emb, tok_emb, gamma, beta, b, s):
  d = gathered.shape[1]
  data_specs = [
      pl.BlockSpec((s, d), lambda i: (0, 0)),
      pl.BlockSpec((s, 1), lambda i: (0, 0)),
      pl.BlockSpec((s, d), lambda i: (0, 0)),
      pl.BlockSpec((2, d), lambda i: (0, 0)),
      pl.BlockSpec((1, d), lambda i: (0, 0)),
      pl.BlockSpec((1, d), lambda i: (0, 0)),
  ]
  operands = (gathered, ttf, pos_emb, tok_emb,
              gamma.reshape(1, d), beta.reshape(1, d))
  common = dict(
      grid=(1,),
      out_specs=pl.BlockSpec((1, s, d), lambda i, _b=bb: (_b, 0, 0)),
      out_shape=jax.ShapeDtypeStruct((b, s, d), jnp.float32),
  )
  if acc is None:
    return pl.pallas_call(_ln_body_first, in_specs=data_specs, **common)(
        *operands)
  return pl.pallas_call(
      _ln_body_acc,
      in_specs=[pl.BlockSpec(memory_space=pl.ANY)] + data_specs,
      input_output_aliases={0: 0},
      **common)(acc, *operands)


def kernel(input_ids, token_type_ids, word_emb, pos_emb, tok_emb,
           ln_gamma, ln_beta):
  b, s = input_ids.shape
  v, d = word_emb.shape
  per_w = s // _NW
  n_ch = per_w // _CH
  ids = input_ids.astype(jnp.int32)
  ttf = token_type_ids.astype(jnp.float32)
  gathered = [
      _sc_gather(ids[bb].reshape(_NW, n_ch, _CH), word_emb, s, d, n_ch)
      for bb in range(b)
  ]
  acc = None
  for bb in range(b):
    acc = _tc_ln_chunk(acc, bb, gathered[bb], ttf[bb].reshape(s, 1),
                       pos_emb, tok_emb, ln_gamma, ln_beta, b, s)
  return acc


# speedup vs baseline: 1.3233x; 1.3233x over previous
import functools

import jax
import jax.numpy as jnp
from jax import lax
from jax.experimental import pallas as pl
from jax.experimental.pallas import tpu as pltpu
from jax.experimental.pallas import tpu_sc as plsc

_NC, _NS = 2, 16
_NW = _NC * _NS
_EPS = 1e-12

_CH = 64
_NBUF = 2


def _sc_gather(ids3, table, n, d, n_ch):
  mesh = plsc.VectorSubcoreMesh(core_axis_name="c", subcore_axis_name="s")
  scratch = [pltpu.VMEM((n_ch, _CH), jnp.int32)]
  scratch += [pltpu.VMEM((_CH, d), jnp.float32) for _ in range(_NBUF)]
  scratch += [pltpu.SemaphoreType.DMA for _ in range(2 * _NBUF)]

  @functools.partial(
      pl.kernel,
      mesh=mesh,
      out_type=jax.ShapeDtypeStruct((n, d), jnp.float32),
      scratch_types=scratch,
  )
  def gather_kernel(ids_hbm, tab_hbm, out_hbm, idx_v, *rest):
    bufs = rest[:_NBUF]
    gsems = rest[_NBUF:2 * _NBUF]
    wsems = rest[2 * _NBUF:]
    wid = lax.axis_index("s") * _NC + lax.axis_index("c")
    base = wid * (n_ch * _CH)
    pltpu.sync_copy(ids_hbm.at[wid], idx_v)

    def start_gather(c):
      return pltpu.async_copy(
          tab_hbm.at[idx_v.at[c]], bufs[c % _NBUF], gsems[c % _NBUF])

    def start_write(c):
      return pltpu.async_copy(
          bufs[c % _NBUF], out_hbm.at[pl.ds(base + c * _CH, _CH)],
          wsems[c % _NBUF])

    gathers = [None] * n_ch
    writes = [None] * n_ch
    for c in range(min(_NBUF, n_ch)):
      gathers[c] = start_gather(c)
    for c in range(n_ch):
      if c >= 1 and c - 1 + _NBUF < n_ch:
        writes[c - 1].wait()
        gathers[c - 1 + _NBUF] = start_gather(c - 1 + _NBUF)
      gathers[c].wait()
      writes[c] = start_write(c)
    for c in range(max(0, n_ch - _NBUF), n_ch):
      writes[c].wait()

  return gather_kernel(ids3, table)


def _ln_body(g_ref, tt_ref, pos_ref, tok_ref, gam_ref, bet_ref, o_ref):
  x = g_ref[...]
  ttf = tt_ref[...]
  tok0 = tok_ref[0, :][None, :]
  tokd = (tok_ref[1, :] - tok_ref[0, :])[None, :]
  x = x + pos_ref[...] + tok0 + ttf * tokd
  mean = jnp.mean(x, axis=1, keepdims=True)
  xc = x - mean
  var = jnp.mean(xc * xc, axis=1, keepdims=True)
  inv = lax.rsqrt(var + _EPS)
  o_ref[0] = xc * inv * gam_ref[...] + bet_ref[...]


_TB = 2048


def _tc_ln(gathered, ttf, pos_emb, tok_emb, gamma, beta, b, s):
  n, d = gathered.shape
  sblk = s // _TB
  grid = (sblk, b)
  return pl.pallas_call(
      _ln_body,
      grid=grid,
      in_specs=[
          pl.BlockSpec((_TB, d), lambda i, j: (j * sblk + i, 0)),
          pl.BlockSpec((_TB, 1), lambda i, j: (j * sblk + i, 0)),
          pl.BlockSpec((_TB, d), lambda i, j: (i, 0)),
          pl.BlockSpec((2, d), lambda i, j: (0, 0)),
          pl.BlockSpec((1, d), lambda i, j: (0, 0)),
          pl.BlockSpec((1, d), lambda i, j: (0, 0)),
      ],
      out_specs=pl.BlockSpec((1, _TB, d), lambda i, j: (j, i, 0)),
      out_shape=jax.ShapeDtypeStruct((b, s, d), jnp.float32),
  )(gathered, ttf, pos_emb, tok_emb, gamma.reshape(1, d), beta.reshape(1, d))


def kernel(input_ids, token_type_ids, word_emb, pos_emb, tok_emb,
           ln_gamma, ln_beta):
  b, s = input_ids.shape
  v, d = word_emb.shape
  n = b * s
  per_w = n // _NW
  n_ch = per_w // _CH
  ids3 = input_ids.reshape(_NW, n_ch, _CH).astype(jnp.int32)
  gathered = _sc_gather(ids3, word_emb, n, d, n_ch)
  ttf = token_type_ids.reshape(n, 1).astype(jnp.float32)
  return _tc_ln(gathered, ttf, pos_emb, tok_emb, ln_gamma, ln_beta, b, s)


# speedup vs baseline: 1.3334x; 1.0076x over previous
import functools

import jax
import jax.numpy as jnp
from jax import lax
from jax.experimental import pallas as pl
from jax.experimental.pallas import tpu as pltpu
from jax.experimental.pallas import tpu_sc as plsc

_NC, _NS = 2, 16
_NW = _NC * _NS
_EPS = 1e-12

_CH = 32
_NBUF = 4


def _sc_gather(ids3, table, n, d, n_ch):
  mesh = plsc.VectorSubcoreMesh(core_axis_name="c", subcore_axis_name="s")
  scratch = [pltpu.VMEM((n_ch, _CH), jnp.int32)]
  scratch += [pltpu.VMEM((_CH, d), jnp.float32) for _ in range(_NBUF)]
  scratch += [pltpu.SemaphoreType.DMA for _ in range(2 * _NBUF)]

  @functools.partial(
      pl.kernel,
      mesh=mesh,
      out_type=jax.ShapeDtypeStruct((n, d), jnp.float32),
      scratch_types=scratch,
  )
  def gather_kernel(ids_hbm, tab_hbm, out_hbm, idx_v, *rest):
    bufs = rest[:_NBUF]
    gsems = rest[_NBUF:2 * _NBUF]
    wsems = rest[2 * _NBUF:]
    wid = lax.axis_index("s") * _NC + lax.axis_index("c")
    base = wid * (n_ch * _CH)
    pltpu.sync_copy(ids_hbm.at[wid], idx_v)

    def start_gather(c):
      return pltpu.async_copy(
          tab_hbm.at[idx_v.at[c]], bufs[c % _NBUF], gsems[c % _NBUF])

    def start_write(c):
      return pltpu.async_copy(
          bufs[c % _NBUF], out_hbm.at[pl.ds(base + c * _CH, _CH)],
          wsems[c % _NBUF])

    gathers = [None] * n_ch
    writes = [None] * n_ch
    for c in range(min(_NBUF, n_ch)):
      gathers[c] = start_gather(c)
    for c in range(n_ch):
      if c >= 1 and c - 1 + _NBUF < n_ch:
        writes[c - 1].wait()
        gathers[c - 1 + _NBUF] = start_gather(c - 1 + _NBUF)
      gathers[c].wait()
      writes[c] = start_write(c)
    for c in range(max(0, n_ch - _NBUF), n_ch):
      writes[c].wait()

  return gather_kernel(ids3, table)


def _ln_body(g_ref, tt_ref, pos_ref, tok_ref, gam_ref, bet_ref, o_ref):
  x = g_ref[...]
  ttf = tt_ref[...]
  tok0 = tok_ref[0, :][None, :]
  tokd = (tok_ref[1, :] - tok_ref[0, :])[None, :]
  x = x + pos_ref[...] + tok0 + ttf * tokd
  mean = jnp.mean(x, axis=1, keepdims=True)
  xc = x - mean
  var = jnp.mean(xc * xc, axis=1, keepdims=True)
  inv = lax.rsqrt(var + _EPS)
  o_ref[0] = xc * inv * gam_ref[...] + bet_ref[...]


_TB = 2048


def _tc_ln(gathered, ttf, pos_emb, tok_emb, gamma, beta, b, s):
  n, d = gathered.shape
  sblk = s // _TB
  grid = (sblk, b)
  return pl.pallas_call(
      _ln_body,
      grid=grid,
      in_specs=[
          pl.BlockSpec((_TB, d), lambda i, j: (j * sblk + i, 0)),
          pl.BlockSpec((_TB, 1), lambda i, j: (j * sblk + i, 0)),
          pl.BlockSpec((_TB, d), lambda i, j: (i, 0)),
          pl.BlockSpec((2, d), lambda i, j: (0, 0)),
          pl.BlockSpec((1, d), lambda i, j: (0, 0)),
          pl.BlockSpec((1, d), lambda i, j: (0, 0)),
      ],
      out_specs=pl.BlockSpec((1, _TB, d), lambda i, j: (j, i, 0)),
      out_shape=jax.ShapeDtypeStruct((b, s, d), jnp.float32),
  )(gathered, ttf, pos_emb, tok_emb, gamma.reshape(1, d), beta.reshape(1, d))


def kernel(input_ids, token_type_ids, word_emb, pos_emb, tok_emb,
           ln_gamma, ln_beta):
  b, s = input_ids.shape
  v, d = word_emb.shape
  n = b * s
  per_w = n // _NW
  n_ch = per_w // _CH
  ids3 = input_ids.reshape(_NW, n_ch, _CH).astype(jnp.int32)
  gathered = _sc_gather(ids3, word_emb, n, d, n_ch)
  ttf = token_type_ids.reshape(n, 1).astype(jnp.float32)
  return _tc_ln(gathered, ttf, pos_emb, tok_emb, ln_gamma, ln_beta, b, s)


# speedup vs baseline: 1.3476x; 1.0106x over previous
import functools

import jax
import jax.numpy as jnp
from jax import lax
from jax.experimental import pallas as pl
from jax.experimental.pallas import tpu as pltpu
from jax.experimental.pallas import tpu_sc as plsc

_NC, _NS = 2, 16
_NW = _NC * _NS
_EPS = 1e-12

_CH = 32
_NBUF = 4


def _sc_gather(ids, table, n, d, n_ch):
  mesh = plsc.VectorSubcoreMesh(core_axis_name="c", subcore_axis_name="s")
  per_w = n_ch * _CH
  w_per_row = ids.shape[1] // per_w
  scratch = [pltpu.VMEM((1, per_w), jnp.int32)]
  scratch += [pltpu.VMEM((_CH, d), jnp.float32) for _ in range(_NBUF)]
  scratch += [pltpu.SemaphoreType.DMA for _ in range(2 * _NBUF)]

  @functools.partial(
      pl.kernel,
      mesh=mesh,
      out_type=jax.ShapeDtypeStruct((n, d), jnp.float32),
      scratch_types=scratch,
  )
  def gather_kernel(ids_hbm, tab_hbm, out_hbm, idx_v, *rest):
    bufs = rest[:_NBUF]
    gsems = rest[_NBUF:2 * _NBUF]
    wsems = rest[2 * _NBUF:]
    wid = lax.axis_index("s") * _NC + lax.axis_index("c")
    base = wid * per_w
    row = wid // w_per_row
    col = (wid % w_per_row) * per_w
    pltpu.sync_copy(
        ids_hbm.at[pl.ds(row, 1), pl.ds(col, per_w)], idx_v)

    def start_gather(c):
      return pltpu.async_copy(
          tab_hbm.at[idx_v.at[0, pl.ds(c * _CH, _CH)]],
          bufs[c % _NBUF], gsems[c % _NBUF])

    def start_write(c):
      return pltpu.async_copy(
          bufs[c % _NBUF], out_hbm.at[pl.ds(base + c * _CH, _CH)],
          wsems[c % _NBUF])

    gathers = [None] * n_ch
    writes = [None] * n_ch
    for c in range(min(_NBUF, n_ch)):
      gathers[c] = start_gather(c)
    for c in range(n_ch):
      if c >= 1 and c - 1 + _NBUF < n_ch:
        writes[c - 1].wait()
        gathers[c - 1 + _NBUF] = start_gather(c - 1 + _NBUF)
      gathers[c].wait()
      writes[c] = start_write(c)
    for c in range(max(0, n_ch - _NBUF), n_ch):
      writes[c].wait()

  return gather_kernel(ids, table)


def _ln_body(g_ref, tt_ref, pos_ref, tok_ref, gam_ref, bet_ref, o_ref):
  x = g_ref[...]
  ttf = tt_ref[...]
  tok0 = tok_ref[0, :][None, :]
  tokd = (tok_ref[1, :] - tok_ref[0, :])[None, :]
  x = x + pos_ref[...] + tok0 + ttf * tokd
  mean = jnp.mean(x, axis=1, keepdims=True)
  xc = x - mean
  var = jnp.mean(xc * xc, axis=1, keepdims=True)
  inv = lax.rsqrt(var + _EPS)
  o_ref[0] = xc * inv * gam_ref[...] + bet_ref[...]


_TB = 2048


def _tc_ln(gathered, ttf, pos_emb, tok_emb, gamma, beta, b, s):
  n, d = gathered.shape
  sblk = s // _TB
  grid = (sblk, b)
  return pl.pallas_call(
      _ln_body,
      grid=grid,
      in_specs=[
          pl.BlockSpec((_TB, d), lambda i, j: (j * sblk + i, 0)),
          pl.BlockSpec((_TB, 1), lambda i, j: (j * sblk + i, 0)),
          pl.BlockSpec((_TB, d), lambda i, j: (i, 0)),
          pl.BlockSpec((2, d), lambda i, j: (0, 0)),
          pl.BlockSpec((1, d), lambda i, j: (0, 0)),
          pl.BlockSpec((1, d), lambda i, j: (0, 0)),
      ],
      out_specs=pl.BlockSpec((1, _TB, d), lambda i, j: (j, i, 0)),
      out_shape=jax.ShapeDtypeStruct((b, s, d), jnp.float32),
  )(gathered, ttf, pos_emb, tok_emb, gamma.reshape(1, d), beta.reshape(1, d))


def kernel(input_ids, token_type_ids, word_emb, pos_emb, tok_emb,
           ln_gamma, ln_beta):
  b, s = input_ids.shape
  v, d = word_emb.shape
  n = b * s
  per_w = n // _NW
  n_ch = per_w // _CH
  gathered = _sc_gather(input_ids.astype(jnp.int32), word_emb, n, d, n_ch)
  ttf = token_type_ids.reshape(n, 1).astype(jnp.float32)
  return _tc_ln(gathered, ttf, pos_emb, tok_emb, ln_gamma, ln_beta, b, s)
